# Initial kernel scaffold; baseline (speedup 1.0000x reference)
import functools

import jax
import jax.numpy as jnp
from jax import lax
from jax.experimental import pallas as pl
from jax.experimental.pallas import tpu as pltpu
from jax.experimental.pallas import tpu_sc as plsc

DELTA_V = 0.2
DELTA_D = 0.2
ALPHA = 1.0
BETA = 1.0
GAMMA = 0.001
K = 16
SLOTS = 34
SEG = K * SLOTS
NLANE = 16
NBANK = NLANE
NW = 32
CHUNK = 1024


def _sc_body(nchunk, feat, lbl, out, acc, red, fb0, fb1, lb0, lb1,
             s0, s1, s2, s3):
    cid = lax.axis_index("c")
    sid = lax.axis_index("s")
    wid = sid * 2 + cid
    wpi = NW // feat.shape[0]
    ppw = nchunk * CHUNK
    n = wid // wpi
    base = (wid % wpi) * ppw

    zeros = jnp.zeros((NLANE,), jnp.float32)
    ones = jnp.ones((NLANE,), jnp.float32)
    lane = lax.iota(jnp.int32, NLANE)
    lane_base = lane * SEG

    def zero_body(j, c):
        acc[pl.ds(j * NLANE, NLANE)] = zeros
        return c
    lax.fori_loop(0, (NBANK * SEG) // NLANE, zero_body, 0)

    def start(idx, fb, lb, sf, sl):
        pltpu.async_copy(feat.at[n, :, pl.ds(base + idx * CHUNK, CHUNK)], fb, sf)
        pltpu.async_copy(lbl.at[n, pl.ds(base + idx * CHUNK, CHUNK)], lb, sl)

    def wait(fb, lb, sf, sl):
        pltpu.make_async_copy(feat.at[n, :, pl.ds(base, CHUNK)], fb, sf).wait()
        pltpu.make_async_copy(lbl.at[n, pl.ds(base, CHUNK)], lb, sl).wait()

    def compute(fb, lb):
        def group(g, c):
            off = g * NLANE
            lab = lb[pl.ds(off, NLANE)]
            idx0 = lane_base + lab * SLOTS
            sq = zeros
            for ch in range(feat.shape[1]):
                v = fb[ch, pl.ds(off, NLANE)]
                plsc.addupdate_scatter(acc, [idx0 + ch], v)
                sq = sq + v * v
            plsc.addupdate_scatter(acc, [idx0 + 32], sq)
            plsc.addupdate_scatter(acc, [idx0 + 33], ones)
            return c
        lax.fori_loop(0, CHUNK // NLANE, group, 0)

    start(0, fb0, lb0, s0, s1)
    start(1, fb1, lb1, s2, s3)

    def step(t, c):
        i0 = t * 2
        wait(fb0, lb0, s0, s1)
        compute(fb0, lb0)

        @pl.when(i0 + 2 < nchunk)
        def _():
            start(i0 + 2, fb0, lb0, s0, s1)

        wait(fb1, lb1, s2, s3)
        compute(fb1, lb1)

        @pl.when(i0 + 3 < nchunk)
        def _():
            start(i0 + 3, fb1, lb1, s2, s3)
        return c
    lax.fori_loop(0, nchunk // 2, step, 0)

    def fold(j, c):
        v = acc[pl.ds(j * NLANE, NLANE)]
        for b in range(1, NBANK):
            v = v + acc[pl.ds(b * SEG + j * NLANE, NLANE)]
        red[pl.ds(j * NLANE, NLANE)] = v
        return c
    lax.fori_loop(0, SEG // NLANE, fold, 0)
    pltpu.sync_copy(red, out.at[wid])


def _epilogue_body(p_ref, var_ref, dist_ref, norm_ref, tot_ref, mu_ref):
    p = p_ref[...]
    N = mu_ref.shape[0]
    wpi = NW // N
    var_acc = jnp.zeros((K, 1), jnp.float32)
    norm_acc = jnp.zeros((K, 1), jnp.float32)
    hs = jnp.zeros((K, K), jnp.float32)
    for n in range(N):
        pn = p[n * wpi]
        for j in range(1, wpi):
            pn = pn + p[n * wpi + j]
        sums = pn[:, :32]
        sumsq = pn[:, 32:33]
        cnt = jnp.maximum(pn[:, 33:34], 1.0)
        mu = sums / cnt
        mu_ref[n, :, :] = mu
        musq = jnp.sum(mu * mu, axis=1, keepdims=True)
        seg_sq = sumsq - cnt * musq
        mse = seg_sq / (32.0 * cnt)
        var_acc = var_acc + jnp.maximum(mse - DELTA_V, 0.0)
        norm_acc = norm_acc + jnp.sqrt(musq + 1e-12)
        diff = mu[:, None, :] - mu[None, :, :]
        d = jnp.sqrt(jnp.sum(diff * diff, axis=2) + 1e-12)
        hs = hs + jnp.maximum(2.0 * DELTA_D - d, 0.0)

    denom = float(N * K)
    var = jnp.sum(var_acc) / denom
    norm = jnp.sum(norm_acc) / denom
    kk = lax.broadcasted_iota(jnp.int32, (K, K), 0)
    ii = lax.broadcasted_iota(jnp.int32, (K, K), 1)
    shift = (kk == ii + 1).astype(jnp.float32)
    hshift = jnp.dot(hs, shift, preferred_element_type=jnp.float32)
    m1 = (kk > ii).astype(jnp.float32)
    m2 = (kk <= ii).astype(jnp.float32)
    dl = jnp.sum(hs * m1 + hshift * m2, axis=0, keepdims=True) / denom

    var_ref[...] = jnp.reshape(var, (1, 1))
    norm_ref[...] = jnp.reshape(norm, (1, 1))
    dist_ref[...] = dl
    tot_ref[...] = ALPHA * var + BETA * dl + GAMMA * norm


def kernel(features, ground_truth):
    N, C, H, W = features.shape
    P = H * W
    wpi = NW // N
    nchunk = P // (wpi * CHUNK)
    f3 = features.reshape(N, C, P)
    labels = ground_truth.reshape(N, P)

    mesh = plsc.VectorSubcoreMesh(core_axis_name="c", subcore_axis_name="s")
    partials = pl.kernel(
        functools.partial(_sc_body, nchunk),
        out_type=jax.ShapeDtypeStruct((NW, SEG), jnp.float32),
        mesh=mesh,
        scratch_types=[
            pltpu.VMEM((NBANK * SEG,), jnp.float32),
            pltpu.VMEM((SEG,), jnp.float32),
            pltpu.VMEM((C, CHUNK), jnp.float32),
            pltpu.VMEM((C, CHUNK), jnp.float32),
            pltpu.VMEM((CHUNK,), jnp.int32),
            pltpu.VMEM((CHUNK,), jnp.int32),
            pltpu.SemaphoreType.DMA,
            pltpu.SemaphoreType.DMA,
            pltpu.SemaphoreType.DMA,
            pltpu.SemaphoreType.DMA,
        ],
    )(f3, labels)

    p3 = partials.reshape(NW, K, SLOTS)
    var, dist, norm, tot, mu = pl.pallas_call(
        _epilogue_body,
        out_shape=(
            jax.ShapeDtypeStruct((1, 1), jnp.float32),
            jax.ShapeDtypeStruct((1, K), jnp.float32),
            jax.ShapeDtypeStruct((1, 1), jnp.float32),
            jax.ShapeDtypeStruct((1, K), jnp.float32),
            jax.ShapeDtypeStruct((N, K, C), jnp.float32),
        ),
    )(p3)

    total_loss = tot[0, : K - 1]
    variance_loss = var[0, 0]
    distance_loss = dist[0, : K - 1]
    normalization_loss = norm[0, 0]
    cluster_mean = jnp.swapaxes(mu, 1, 2)
    return (total_loss, (variance_loss, distance_loss,
                         normalization_loss, cluster_mean))


# speedup vs baseline: 16.7991x; 16.7991x over previous
import functools

import jax
import jax.numpy as jnp
from jax import lax
from jax.experimental import pallas as pl
from jax.experimental.pallas import tpu as pltpu
from jax.experimental.pallas import tpu_sc as plsc

DELTA_V = 0.2
DELTA_D = 0.2
ALPHA = 1.0
BETA = 1.0
GAMMA = 0.001
K = 16
SLOTS = 34
SEG = K * SLOTS
NLANE = 16
NBANK = NLANE
NW = 32
CHUNK = 1024


def _sc_body(nchunk, feat, lbl, out, acc, red, fb0, fb1, lb0, lb1,
             s0, s1, s2, s3):
    cid = lax.axis_index("c")
    sid = lax.axis_index("s")
    wid = sid * 2 + cid
    wpi = NW // feat.shape[0]
    ppw = nchunk * CHUNK
    n = wid // wpi
    base = (wid % wpi) * ppw

    zeros = jnp.zeros((NLANE,), jnp.float32)
    ones = jnp.ones((NLANE,), jnp.float32)
    lane = lax.iota(jnp.int32, NLANE)
    lane_base = lane * SEG

    def zero_body(j, c):
        acc[pl.ds(j * NLANE, NLANE)] = zeros
        return c
    lax.fori_loop(0, (NBANK * SEG) // NLANE, zero_body, 0)

    def start(idx, fb, lb, sf, sl):
        pltpu.async_copy(feat.at[n, :, pl.ds(base + idx * CHUNK, CHUNK)], fb, sf)
        pltpu.async_copy(lbl.at[n, pl.ds(base + idx * CHUNK, CHUNK)], lb, sl)

    def wait(fb, lb, sf, sl):
        pltpu.make_async_copy(feat.at[n, :, pl.ds(base, CHUNK)], fb, sf).wait()
        pltpu.make_async_copy(lbl.at[n, pl.ds(base, CHUNK)], lb, sl).wait()

    def compute(fb, lb):
        def group(g, c):
            off = g * NLANE
            lab = lb[pl.ds(off, NLANE)]
            idx0 = lane_base + lab * SLOTS
            sq = zeros
            for ch in range(feat.shape[1]):
                v = fb[ch, pl.ds(off, NLANE)]
                plsc.addupdate_scatter(acc, [idx0 + ch], v)
                sq = sq + v * v
            plsc.addupdate_scatter(acc, [idx0 + 32], sq)
            plsc.addupdate_scatter(acc, [idx0 + 33], ones)
            return c
        lax.fori_loop(0, CHUNK // NLANE, group, 0)

    start(0, fb0, lb0, s0, s1)
    start(1, fb1, lb1, s2, s3)

    def step(t, c):
        i0 = t * 2
        wait(fb0, lb0, s0, s1)
        compute(fb0, lb0)

        @pl.when(i0 + 2 < nchunk)
        def _():
            start(i0 + 2, fb0, lb0, s0, s1)

        wait(fb1, lb1, s2, s3)
        compute(fb1, lb1)

        @pl.when(i0 + 3 < nchunk)
        def _():
            start(i0 + 3, fb1, lb1, s2, s3)
        return c
    lax.fori_loop(0, nchunk // 2, step, 0)

    def fold(j, c):
        v = acc[pl.ds(j * NLANE, NLANE)]
        for b in range(1, NBANK):
            v = v + acc[pl.ds(b * SEG + j * NLANE, NLANE)]
        red[pl.ds(j * NLANE, NLANE)] = v
        return c
    lax.fori_loop(0, SEG // NLANE, fold, 0)
    pltpu.sync_copy(red, out.at[wid])


def _epilogue_body(p_ref, var_ref, dist_ref, norm_ref, tot_ref, mu_ref):
    p = p_ref[...]
    N = mu_ref.shape[0]
    wpi = NW // N
    var_acc = jnp.zeros((K, 1), jnp.float32)
    norm_acc = jnp.zeros((K, 1), jnp.float32)
    hs = jnp.zeros((K, K), jnp.float32)
    for n in range(N):
        pn = p[n * wpi]
        for j in range(1, wpi):
            pn = pn + p[n * wpi + j]
        sums = pn[:, :32]
        sumsq = pn[:, 32:33]
        cnt = jnp.maximum(pn[:, 33:34], 1.0)
        mu = sums / cnt
        mu_ref[n, :, :] = mu
        musq = jnp.sum(mu * mu, axis=1, keepdims=True)
        seg_sq = sumsq - cnt * musq
        mse = seg_sq / (32.0 * cnt)
        var_acc = var_acc + jnp.maximum(mse - DELTA_V, 0.0)
        norm_acc = norm_acc + jnp.sqrt(musq + 1e-12)
        diff = mu[:, None, :] - mu[None, :, :]
        d = jnp.sqrt(jnp.sum(diff * diff, axis=2) + 1e-12)
        hs = hs + jnp.maximum(2.0 * DELTA_D - d, 0.0)

    denom = float(N * K)
    var = jnp.sum(var_acc) / denom
    norm = jnp.sum(norm_acc) / denom
    kk = lax.broadcasted_iota(jnp.int32, (K, K), 0)
    ii = lax.broadcasted_iota(jnp.int32, (K, K), 1)
    shift = (kk == ii + 1).astype(jnp.float32)
    hshift = jnp.dot(hs, shift, preferred_element_type=jnp.float32)
    m1 = (kk > ii).astype(jnp.float32)
    m2 = (kk <= ii).astype(jnp.float32)
    dl = jnp.sum(hs * m1 + hshift * m2, axis=0, keepdims=True) / denom

    var_ref[...] = jnp.reshape(var, (1, 1))
    norm_ref[...] = jnp.reshape(norm, (1, 1))
    dist_ref[...] = dl
    tot_ref[...] = ALPHA * var + BETA * dl + GAMMA * norm


def kernel(features, ground_truth):
    N, C, H, W = features.shape
    P = H * W
    wpi = NW // N
    nchunk = P // (wpi * CHUNK)
    f3 = features.reshape(N, C, P)
    labels = ground_truth.reshape(N, P)

    mesh = plsc.VectorSubcoreMesh(core_axis_name="c", subcore_axis_name="s")
    partials = pl.kernel(
        functools.partial(_sc_body, nchunk),
        out_type=jax.ShapeDtypeStruct((NW, SEG), jnp.float32),
        mesh=mesh,
        compiler_params=pltpu.CompilerParams(needs_layout_passes=False),
        scratch_types=[
            pltpu.VMEM((NBANK * SEG,), jnp.float32),
            pltpu.VMEM((SEG,), jnp.float32),
            pltpu.VMEM((C, CHUNK), jnp.float32),
            pltpu.VMEM((C, CHUNK), jnp.float32),
            pltpu.VMEM((CHUNK,), jnp.int32),
            pltpu.VMEM((CHUNK,), jnp.int32),
            pltpu.SemaphoreType.DMA,
            pltpu.SemaphoreType.DMA,
            pltpu.SemaphoreType.DMA,
            pltpu.SemaphoreType.DMA,
        ],
    )(f3, labels)

    p3 = partials.reshape(NW, K, SLOTS)
    var, dist, norm, tot, mu = pl.pallas_call(
        _epilogue_body,
        out_shape=(
            jax.ShapeDtypeStruct((1, 1), jnp.float32),
            jax.ShapeDtypeStruct((1, K), jnp.float32),
            jax.ShapeDtypeStruct((1, 1), jnp.float32),
            jax.ShapeDtypeStruct((1, K), jnp.float32),
            jax.ShapeDtypeStruct((N, K, C), jnp.float32),
        ),
    )(p3)

    total_loss = tot[0, : K - 1]
    variance_loss = var[0, 0]
    distance_loss = dist[0, : K - 1]
    normalization_loss = norm[0, 0]
    cluster_mean = jnp.swapaxes(mu, 1, 2)
    return (total_loss, (variance_loss, distance_loss,
                         normalization_loss, cluster_mean))


# speedup vs baseline: 19.2239x; 1.1443x over previous
import functools

import jax
import jax.numpy as jnp
from jax import lax
from jax.experimental import pallas as pl
from jax.experimental.pallas import tpu as pltpu
from jax.experimental.pallas import tpu_sc as plsc

DELTA_V = 0.2
DELTA_D = 0.2
ALPHA = 1.0
BETA = 1.0
GAMMA = 0.001
K = 16
SLOTS = 48
SEG = K * SLOTS
NLANE = 16
NBANK = NLANE
BSTRIDE = SEG + 1
ACCW = NBANK * BSTRIDE + NLANE
NW = 32
CHUNK = 1024


def _sc_body(nchunk, feat, lbl, out, acc, red, fb0, fb1, lb0, lb1,
             s0, s1, s2, s3):
    cid = lax.axis_index("c")
    sid = lax.axis_index("s")
    wid = sid * 2 + cid
    wpi = NW // feat.shape[0]
    ppw = nchunk * CHUNK
    n = wid // wpi
    base = (wid % wpi) * ppw

    zeros = jnp.zeros((NLANE,), jnp.float32)
    ones = jnp.ones((NLANE,), jnp.float32)
    lane = lax.iota(jnp.int32, NLANE)
    lane_base = lane * BSTRIDE

    def zero_body(j, c):
        acc[pl.ds(j * NLANE, NLANE)] = zeros
        return c
    lax.fori_loop(0, ACCW // NLANE, zero_body, 0)

    def start(idx, fb, lb, sf, sl):
        pltpu.async_copy(feat.at[n, :, pl.ds(base + idx * CHUNK, CHUNK)], fb, sf)
        pltpu.async_copy(lbl.at[n, pl.ds(base + idx * CHUNK, CHUNK)], lb, sl)

    def wait(fb, lb, sf, sl):
        pltpu.make_async_copy(feat.at[n, :, pl.ds(base, CHUNK)], fb, sf).wait()
        pltpu.make_async_copy(lbl.at[n, pl.ds(base, CHUNK)], lb, sl).wait()

    def compute(fb, lb):
        def group(g, c):
            off = g * NLANE
            lab = lb[pl.ds(off, NLANE)]
            idx0 = lane_base + lab * SLOTS
            sq = [zeros, zeros, zeros, zeros]
            for ch in range(feat.shape[1]):
                v = fb[ch, pl.ds(off, NLANE)]
                plsc.addupdate_scatter(acc, [idx0 + ch], v)
                sq[ch % 4] = sq[ch % 4] + v * v
            plsc.addupdate_scatter(acc, [idx0 + 32], (sq[0] + sq[1]) + (sq[2] + sq[3]))
            plsc.addupdate_scatter(acc, [idx0 + 33], ones)
            return c
        lax.fori_loop(0, CHUNK // NLANE, group, 0)

    start(0, fb0, lb0, s0, s1)
    start(1, fb1, lb1, s2, s3)

    def step(t, c):
        i0 = t * 2
        wait(fb0, lb0, s0, s1)
        compute(fb0, lb0)

        @pl.when(i0 + 2 < nchunk)
        def _():
            start(i0 + 2, fb0, lb0, s0, s1)

        wait(fb1, lb1, s2, s3)
        compute(fb1, lb1)

        @pl.when(i0 + 3 < nchunk)
        def _():
            start(i0 + 3, fb1, lb1, s2, s3)
        return c
    lax.fori_loop(0, nchunk // 2, step, 0)

    def fold(j, c):
        v = acc[pl.ds(j * NLANE, NLANE)]
        for b in range(1, NBANK):
            v = v + acc[pl.ds(b * BSTRIDE + j * NLANE, NLANE)]
        red[pl.ds(j * NLANE, NLANE)] = v
        return c
    lax.fori_loop(0, SEG // NLANE, fold, 0)
    pltpu.sync_copy(red, out.at[wid])


def _epilogue_body(p_ref, var_ref, dist_ref, norm_ref, tot_ref, mu_ref):
    p = p_ref[...]
    N = mu_ref.shape[0]
    wpi = NW // N
    var_acc = jnp.zeros((K, 1), jnp.float32)
    norm_acc = jnp.zeros((K, 1), jnp.float32)
    hs = jnp.zeros((K, K), jnp.float32)
    for n in range(N):
        pn = p[n * wpi]
        for j in range(1, wpi):
            pn = pn + p[n * wpi + j]
        sums = pn[:, :32]
        sumsq = pn[:, 32:33]
        cnt = jnp.maximum(pn[:, 33:34], 1.0)
        mu = sums / cnt
        mu_ref[n, :, :] = mu
        musq = jnp.sum(mu * mu, axis=1, keepdims=True)
        seg_sq = sumsq - cnt * musq
        mse = seg_sq / (32.0 * cnt)
        var_acc = var_acc + jnp.maximum(mse - DELTA_V, 0.0)
        norm_acc = norm_acc + jnp.sqrt(musq + 1e-12)
        diff = mu[:, None, :] - mu[None, :, :]
        d = jnp.sqrt(jnp.sum(diff * diff, axis=2) + 1e-12)
        hs = hs + jnp.maximum(2.0 * DELTA_D - d, 0.0)

    denom = float(N * K)
    var = jnp.sum(var_acc) / denom
    norm = jnp.sum(norm_acc) / denom
    kk = lax.broadcasted_iota(jnp.int32, (K, K), 0)
    ii = lax.broadcasted_iota(jnp.int32, (K, K), 1)
    shift = (kk == ii + 1).astype(jnp.float32)
    hshift = jnp.dot(hs, shift, preferred_element_type=jnp.float32)
    m1 = (kk > ii).astype(jnp.float32)
    m2 = (kk <= ii).astype(jnp.float32)
    dl = jnp.sum(hs * m1 + hshift * m2, axis=0, keepdims=True) / denom

    var_ref[...] = jnp.reshape(var, (1, 1))
    norm_ref[...] = jnp.reshape(norm, (1, 1))
    dist_ref[...] = dl
    tot_ref[...] = ALPHA * var + BETA * dl + GAMMA * norm


def kernel(features, ground_truth):
    N, C, H, W = features.shape
    P = H * W
    wpi = NW // N
    nchunk = P // (wpi * CHUNK)
    f3 = features.reshape(N, C, P)
    labels = ground_truth.reshape(N, P)

    mesh = plsc.VectorSubcoreMesh(core_axis_name="c", subcore_axis_name="s")
    partials = pl.kernel(
        functools.partial(_sc_body, nchunk),
        out_type=jax.ShapeDtypeStruct((NW, SEG), jnp.float32),
        mesh=mesh,
        compiler_params=pltpu.CompilerParams(needs_layout_passes=False),
        scratch_types=[
            pltpu.VMEM((ACCW,), jnp.float32),
            pltpu.VMEM((SEG,), jnp.float32),
            pltpu.VMEM((C, CHUNK), jnp.float32),
            pltpu.VMEM((C, CHUNK), jnp.float32),
            pltpu.VMEM((CHUNK,), jnp.int32),
            pltpu.VMEM((CHUNK,), jnp.int32),
            pltpu.SemaphoreType.DMA,
            pltpu.SemaphoreType.DMA,
            pltpu.SemaphoreType.DMA,
            pltpu.SemaphoreType.DMA,
        ],
    )(f3, labels)

    p3 = partials.reshape(NW, K, SLOTS)
    var, dist, norm, tot, mu = pl.pallas_call(
        _epilogue_body,
        out_shape=(
            jax.ShapeDtypeStruct((1, 1), jnp.float32),
            jax.ShapeDtypeStruct((1, K), jnp.float32),
            jax.ShapeDtypeStruct((1, 1), jnp.float32),
            jax.ShapeDtypeStruct((1, K), jnp.float32),
            jax.ShapeDtypeStruct((N, K, C), jnp.float32),
        ),
    )(p3)

    total_loss = tot[0, : K - 1]
    variance_loss = var[0, 0]
    distance_loss = dist[0, : K - 1]
    normalization_loss = norm[0, 0]
    cluster_mean = jnp.swapaxes(mu, 1, 2)
    return (total_loss, (variance_loss, distance_loss,
                         normalization_loss, cluster_mean))


# speedup vs baseline: 26.7728x; 1.3927x over previous
import functools

import jax
import jax.numpy as jnp
from jax import lax
from jax.experimental import pallas as pl
from jax.experimental.pallas import tpu as pltpu
from jax.experimental.pallas import tpu_sc as plsc

DELTA_V = 0.2
DELTA_D = 0.2
ALPHA = 1.0
BETA = 1.0
GAMMA = 0.001
K = 16
SLOTS = 48
SEG = K * SLOTS
NLANE = 16
NBANK = NLANE
BSTRIDE = SEG + 1
ACCW = NBANK * BSTRIDE + NLANE
NW = 32
CHUNK = 1024


def _sc_body(nchunk, feat, lbl, out, acc, red, fb0, fb1, lb0, lb1,
             s0, s1, s2, s3):
    cid = lax.axis_index("c")
    sid = lax.axis_index("s")
    wid = sid * 2 + cid
    wpi = NW // feat.shape[0]
    ppw = nchunk * CHUNK
    n = wid // wpi
    base = (wid % wpi) * ppw

    zeros = jnp.zeros((NLANE,), jnp.float32)
    ones = jnp.ones((NLANE,), jnp.float32)
    lane = lax.iota(jnp.int32, NLANE)
    lane_base = lane * BSTRIDE

    def zero_body(j, c):
        acc[pl.ds(j * NLANE, NLANE)] = zeros
        return c
    lax.fori_loop(0, ACCW // NLANE, zero_body, 0)

    def start(idx, fb, lb, sf, sl):
        pltpu.async_copy(feat.at[n, :, pl.ds(base + idx * CHUNK, CHUNK)], fb, sf)
        pltpu.async_copy(lbl.at[n, pl.ds(base + idx * CHUNK, CHUNK)], lb, sl)

    def wait(fb, lb, sf, sl):
        pltpu.make_async_copy(feat.at[n, :, pl.ds(base, CHUNK)], fb, sf).wait()
        pltpu.make_async_copy(lbl.at[n, pl.ds(base, CHUNK)], lb, sl).wait()

    def compute(fb, lb):
        nch = feat.shape[1]

        def group(g, c):
            off = g * NLANE
            lab = lb[pl.ds(off, NLANE)]
            idx0 = lane_base + lab * SLOTS
            sq = [zeros, zeros, zeros, zeros]
            for c0 in range(0, nch, 16):
                vs = [fb[ch, pl.ds(off, NLANE)] for ch in range(c0, c0 + 16)]
                for j, ch in enumerate(range(c0, c0 + 16)):
                    sq[ch % 4] = sq[ch % 4] + vs[j] * vs[j]
                for j, ch in enumerate(range(c0, c0 + 16)):
                    plsc.addupdate_scatter(acc, [idx0 + ch], vs[j])
            plsc.addupdate_scatter(acc, [idx0 + 32], (sq[0] + sq[1]) + (sq[2] + sq[3]))
            plsc.addupdate_scatter(acc, [idx0 + 33], ones)
            return c
        lax.fori_loop(0, CHUNK // NLANE, group, 0)

    start(0, fb0, lb0, s0, s1)
    start(1, fb1, lb1, s2, s3)

    def step(t, c):
        i0 = t * 2
        wait(fb0, lb0, s0, s1)
        compute(fb0, lb0)

        @pl.when(i0 + 2 < nchunk)
        def _():
            start(i0 + 2, fb0, lb0, s0, s1)

        wait(fb1, lb1, s2, s3)
        compute(fb1, lb1)

        @pl.when(i0 + 3 < nchunk)
        def _():
            start(i0 + 3, fb1, lb1, s2, s3)
        return c
    lax.fori_loop(0, nchunk // 2, step, 0)

    def fold(j, c):
        v = acc[pl.ds(j * NLANE, NLANE)]
        for b in range(1, NBANK):
            v = v + acc[pl.ds(b * BSTRIDE + j * NLANE, NLANE)]
        red[pl.ds(j * NLANE, NLANE)] = v
        return c
    lax.fori_loop(0, SEG // NLANE, fold, 0)
    pltpu.sync_copy(red, out.at[wid])


def _epilogue_body(p_ref, var_ref, dist_ref, norm_ref, tot_ref, mu_ref):
    p = p_ref[...]
    N = mu_ref.shape[0]
    wpi = NW // N
    var_acc = jnp.zeros((K, 1), jnp.float32)
    norm_acc = jnp.zeros((K, 1), jnp.float32)
    hs = jnp.zeros((K, K), jnp.float32)
    for n in range(N):
        pn = p[n * wpi]
        for j in range(1, wpi):
            pn = pn + p[n * wpi + j]
        sums = pn[:, :32]
        sumsq = pn[:, 32:33]
        cnt = jnp.maximum(pn[:, 33:34], 1.0)
        mu = sums / cnt
        mu_ref[n, :, :] = mu
        musq = jnp.sum(mu * mu, axis=1, keepdims=True)
        seg_sq = sumsq - cnt * musq
        mse = seg_sq / (32.0 * cnt)
        var_acc = var_acc + jnp.maximum(mse - DELTA_V, 0.0)
        norm_acc = norm_acc + jnp.sqrt(musq + 1e-12)
        diff = mu[:, None, :] - mu[None, :, :]
        d = jnp.sqrt(jnp.sum(diff * diff, axis=2) + 1e-12)
        hs = hs + jnp.maximum(2.0 * DELTA_D - d, 0.0)

    denom = float(N * K)
    var = jnp.sum(var_acc) / denom
    norm = jnp.sum(norm_acc) / denom
    kk = lax.broadcasted_iota(jnp.int32, (K, K), 0)
    ii = lax.broadcasted_iota(jnp.int32, (K, K), 1)
    shift = (kk == ii + 1).astype(jnp.float32)
    hshift = jnp.dot(hs, shift, preferred_element_type=jnp.float32)
    m1 = (kk > ii).astype(jnp.float32)
    m2 = (kk <= ii).astype(jnp.float32)
    dl = jnp.sum(hs * m1 + hshift * m2, axis=0, keepdims=True) / denom

    var_ref[...] = jnp.reshape(var, (1, 1))
    norm_ref[...] = jnp.reshape(norm, (1, 1))
    dist_ref[...] = dl
    tot_ref[...] = ALPHA * var + BETA * dl + GAMMA * norm


def kernel(features, ground_truth):
    N, C, H, W = features.shape
    P = H * W
    wpi = NW // N
    nchunk = P // (wpi * CHUNK)
    f3 = features.reshape(N, C, P)
    labels = ground_truth.reshape(N, P)

    mesh = plsc.VectorSubcoreMesh(core_axis_name="c", subcore_axis_name="s")
    partials = pl.kernel(
        functools.partial(_sc_body, nchunk),
        out_type=jax.ShapeDtypeStruct((NW, SEG), jnp.float32),
        mesh=mesh,
        compiler_params=pltpu.CompilerParams(needs_layout_passes=False),
        scratch_types=[
            pltpu.VMEM((ACCW,), jnp.float32),
            pltpu.VMEM((SEG,), jnp.float32),
            pltpu.VMEM((C, CHUNK), jnp.float32),
            pltpu.VMEM((C, CHUNK), jnp.float32),
            pltpu.VMEM((CHUNK,), jnp.int32),
            pltpu.VMEM((CHUNK,), jnp.int32),
            pltpu.SemaphoreType.DMA,
            pltpu.SemaphoreType.DMA,
            pltpu.SemaphoreType.DMA,
            pltpu.SemaphoreType.DMA,
        ],
    )(f3, labels)

    p3 = partials.reshape(NW, K, SLOTS)
    var, dist, norm, tot, mu = pl.pallas_call(
        _epilogue_body,
        out_shape=(
            jax.ShapeDtypeStruct((1, 1), jnp.float32),
            jax.ShapeDtypeStruct((1, K), jnp.float32),
            jax.ShapeDtypeStruct((1, 1), jnp.float32),
            jax.ShapeDtypeStruct((1, K), jnp.float32),
            jax.ShapeDtypeStruct((N, K, C), jnp.float32),
        ),
    )(p3)

    total_loss = tot[0, : K - 1]
    variance_loss = var[0, 0]
    distance_loss = dist[0, : K - 1]
    normalization_loss = norm[0, 0]
    cluster_mean = jnp.swapaxes(mu, 1, 2)
    return (total_loss, (variance_loss, distance_loss,
                         normalization_loss, cluster_mean))


# speedup vs baseline: 63.3228x; 2.3652x over previous
import functools

import jax
import jax.numpy as jnp
from jax import lax
from jax.experimental import pallas as pl
from jax.experimental.pallas import tpu as pltpu
from jax.experimental.pallas import tpu_sc as plsc

DELTA_V = 0.2
DELTA_D = 0.2
ALPHA = 1.0
BETA = 1.0
GAMMA = 0.001
K = 16
SLOTS = 64
SEG = K * SLOTS
NLANE = 16
NBANK = NLANE
BSTRIDE = SEG + 1
ACCW = 16512
NW = 32
TH, TW = 8, 128


def _sc_body(feat, lbl, out, acc, red, fb0, fb1, lb0, lb1,
             s0, s1, s2, s3):
    cid = lax.axis_index("c")
    sid = lax.axis_index("s")
    wid = sid * 2 + cid
    N, C, H, W = feat.shape
    wpi = NW // N
    rows = H // wpi
    ntr, ntc = rows // TH, W // TW
    nchunk = ntr * ntc
    n = wid // wpi
    h_base = (wid % wpi) * rows

    zeros = jnp.zeros((NLANE,), jnp.float32)
    ones = jnp.ones((NLANE,), jnp.float32)
    lane = lax.iota(jnp.int32, NLANE)
    lane_base = lane * BSTRIDE

    def zero_body(j, c):
        acc[pl.ds(j * NLANE, NLANE)] = zeros
        return c
    lax.fori_loop(0, ACCW // NLANE, zero_body, 0)

    def start(idx, fb, lb, sf, sl):
        h0 = h_base + (idx // ntc) * TH
        w0 = (idx % ntc) * TW
        pltpu.async_copy(feat.at[n, :, pl.ds(h0, TH), pl.ds(w0, TW)], fb, sf)
        pltpu.async_copy(lbl.at[n, pl.ds(h0, TH), pl.ds(w0, TW)], lb, sl)

    def wait(fb, lb, sf, sl):
        pltpu.make_async_copy(
            feat.at[n, :, pl.ds(h_base, TH), pl.ds(0, TW)], fb, sf).wait()
        pltpu.make_async_copy(
            lbl.at[n, pl.ds(h_base, TH), pl.ds(0, TW)], lb, sl).wait()

    def compute(fb, lb):
        def group(g, c):
            r = g // (TW // NLANE)
            col = (g % (TW // NLANE)) * NLANE
            lab = lb[r, pl.ds(col, NLANE)]
            idx0 = lane_base + lab * SLOTS
            sq = [zeros, zeros, zeros, zeros]
            for c0 in range(0, C, 16):
                vs = [fb[ch, r, pl.ds(col, NLANE)] for ch in range(c0, c0 + 16)]
                for j, ch in enumerate(range(c0, c0 + 16)):
                    sq[ch % 4] = sq[ch % 4] + vs[j] * vs[j]
                for j, ch in enumerate(range(c0, c0 + 16)):
                    plsc.addupdate_scatter(acc, [idx0 + ch], vs[j])
            plsc.addupdate_scatter(acc, [idx0 + 32], (sq[0] + sq[1]) + (sq[2] + sq[3]))
            plsc.addupdate_scatter(acc, [idx0 + 33], ones)
            return c
        lax.fori_loop(0, (TH * TW) // NLANE, group, 0)

    start(0, fb0, lb0, s0, s1)
    start(1, fb1, lb1, s2, s3)

    def step(t, c):
        i0 = t * 2
        wait(fb0, lb0, s0, s1)
        compute(fb0, lb0)

        @pl.when(i0 + 2 < nchunk)
        def _():
            start(i0 + 2, fb0, lb0, s0, s1)

        wait(fb1, lb1, s2, s3)
        compute(fb1, lb1)

        @pl.when(i0 + 3 < nchunk)
        def _():
            start(i0 + 3, fb1, lb1, s2, s3)
        return c
    lax.fori_loop(0, nchunk // 2, step, 0)

    def fold(j, c):
        v = acc[pl.ds(j * NLANE, NLANE)]
        for b in range(1, NBANK):
            v = v + acc[pl.ds(b * BSTRIDE + j * NLANE, NLANE)]
        red[j // TH, pl.ds((j % TH) * NLANE, NLANE)] = v
        return c
    lax.fori_loop(0, SEG // NLANE, fold, 0)
    pltpu.sync_copy(red, out.at[wid])


def _epilogue_body(p_ref, var_ref, dist_ref, norm_ref, tot_ref, mu_ref):
    p = p_ref[...]
    N = mu_ref.shape[0]
    wpi = NW // N
    var_acc = jnp.zeros((K, 1), jnp.float32)
    norm_acc = jnp.zeros((K, 1), jnp.float32)
    hs = jnp.zeros((K, K), jnp.float32)
    for n in range(N):
        pn = p[n * wpi]
        for j in range(1, wpi):
            pn = pn + p[n * wpi + j]
        sums = pn[:, :32]
        sumsq = pn[:, 32:33]
        cnt = jnp.maximum(pn[:, 33:34], 1.0)
        mu = sums / cnt
        mu_ref[n, :, :] = mu
        musq = jnp.sum(mu * mu, axis=1, keepdims=True)
        seg_sq = sumsq - cnt * musq
        mse = seg_sq / (32.0 * cnt)
        var_acc = var_acc + jnp.maximum(mse - DELTA_V, 0.0)
        norm_acc = norm_acc + jnp.sqrt(musq + 1e-12)
        diff = mu[:, None, :] - mu[None, :, :]
        d = jnp.sqrt(jnp.sum(diff * diff, axis=2) + 1e-12)
        hs = hs + jnp.maximum(2.0 * DELTA_D - d, 0.0)

    denom = float(N * K)
    var = jnp.sum(var_acc) / denom
    norm = jnp.sum(norm_acc) / denom
    kk = lax.broadcasted_iota(jnp.int32, (K, K), 0)
    ii = lax.broadcasted_iota(jnp.int32, (K, K), 1)
    shift = (kk == ii + 1).astype(jnp.float32)
    hshift = jnp.dot(hs, shift, preferred_element_type=jnp.float32)
    m1 = (kk > ii).astype(jnp.float32)
    m2 = (kk <= ii).astype(jnp.float32)
    dl = jnp.sum(hs * m1 + hshift * m2, axis=0, keepdims=True) / denom

    var_ref[...] = jnp.reshape(var, (1, 1))
    norm_ref[...] = jnp.reshape(norm, (1, 1))
    dist_ref[...] = dl
    tot_ref[...] = ALPHA * var + BETA * dl + GAMMA * norm


def kernel(features, ground_truth):
    N, C, H, W = features.shape

    mesh = plsc.VectorSubcoreMesh(core_axis_name="c", subcore_axis_name="s")
    partials = pl.kernel(
        _sc_body,
        out_type=jax.ShapeDtypeStruct((NW, TH, TW), jnp.float32),
        mesh=mesh,
        compiler_params=pltpu.CompilerParams(
            needs_layout_passes=False, use_tc_tiling_on_sc=True),
        scratch_types=[
            pltpu.VMEM((ACCW,), jnp.float32),
            pltpu.VMEM((TH, TW), jnp.float32),
            pltpu.VMEM((C, TH, TW), jnp.float32),
            pltpu.VMEM((C, TH, TW), jnp.float32),
            pltpu.VMEM((TH, TW), jnp.int32),
            pltpu.VMEM((TH, TW), jnp.int32),
            pltpu.SemaphoreType.DMA,
            pltpu.SemaphoreType.DMA,
            pltpu.SemaphoreType.DMA,
            pltpu.SemaphoreType.DMA,
        ],
    )(features, ground_truth)

    p3 = partials.reshape(NW, K, SLOTS)
    var, dist, norm, tot, mu = pl.pallas_call(
        _epilogue_body,
        out_shape=(
            jax.ShapeDtypeStruct((1, 1), jnp.float32),
            jax.ShapeDtypeStruct((1, K), jnp.float32),
            jax.ShapeDtypeStruct((1, 1), jnp.float32),
            jax.ShapeDtypeStruct((1, K), jnp.float32),
            jax.ShapeDtypeStruct((N, K, C), jnp.float32),
        ),
    )(p3)

    total_loss = tot[0, : K - 1]
    variance_loss = var[0, 0]
    distance_loss = dist[0, : K - 1]
    normalization_loss = norm[0, 0]
    cluster_mean = jnp.swapaxes(mu, 1, 2)
    return (total_loss, (variance_loss, distance_loss,
                         normalization_loss, cluster_mean))


# speedup vs baseline: 75.5343x; 1.1928x over previous
import functools

import jax
import jax.numpy as jnp
from jax import lax
from jax.experimental import pallas as pl
from jax.experimental.pallas import tpu as pltpu
from jax.experimental.pallas import tpu_sc as plsc

DELTA_V = 0.2
DELTA_D = 0.2
ALPHA = 1.0
BETA = 1.0
GAMMA = 0.001
K = 16
SLOTS = 64
SEG = K * SLOTS
NLANE = 16
NBANK = NLANE
BSTRIDE = SEG + 1
ACCW = 16512
NW = 32
TH, TW = 8, 128


def _sc_body(feat, lbl, out, acc, red, lbv, fb0, fb1, lb0, lb1,
             s0, s1, s2, s3):
    cid = lax.axis_index("c")
    sid = lax.axis_index("s")
    wid = sid * 2 + cid
    N, C, H, W = feat.shape
    wpi = NW // N
    rows = H // wpi
    ntr, ntc = rows // TH, W // TW
    nchunk = ntr * ntc
    n = wid // wpi
    h_base = (wid % wpi) * rows

    zeros = jnp.zeros((NLANE,), jnp.float32)
    ones = jnp.ones((NLANE,), jnp.float32)
    lane = lax.iota(jnp.int32, NLANE)
    lbv[pl.ds(0, NLANE)] = lane * BSTRIDE

    def zero_body(j, c):
        acc[pl.ds(j * NLANE, NLANE)] = zeros
        return c
    lax.fori_loop(0, ACCW // NLANE, zero_body, 0)

    def start(idx, fb, lb, sf, sl):
        h0 = h_base + (idx // ntc) * TH
        w0 = (idx % ntc) * TW
        pltpu.async_copy(feat.at[n, :, pl.ds(h0, TH), pl.ds(w0, TW)], fb, sf)
        pltpu.async_copy(lbl.at[n, pl.ds(h0, TH), pl.ds(w0, TW)], lb, sl)

    def wait(fb, lb, sf, sl):
        pltpu.make_async_copy(
            feat.at[n, :, pl.ds(h_base, TH), pl.ds(0, TW)], fb, sf).wait()
        pltpu.make_async_copy(
            lbl.at[n, pl.ds(h_base, TH), pl.ds(0, TW)], lb, sl).wait()

    def compute(fb, lb):
        def group(g, c):
            r = g // (TW // NLANE)
            col = (g % (TW // NLANE)) * NLANE
            lab = lb[r, pl.ds(col, NLANE)]
            idx0 = lbv[pl.ds(0, NLANE)] + lab * SLOTS
            sq = [zeros, zeros, zeros, zeros]
            for c0 in range(0, C, 16):
                vs = [fb[ch, r, pl.ds(col, NLANE)] for ch in range(c0, c0 + 16)]
                for j, ch in enumerate(range(c0, c0 + 16)):
                    sq[ch % 4] = sq[ch % 4] + vs[j] * vs[j]
                for j, ch in enumerate(range(c0, c0 + 16)):
                    plsc.addupdate_scatter(acc, [idx0 + ch], vs[j])
            plsc.addupdate_scatter(acc, [idx0 + 32], (sq[0] + sq[1]) + (sq[2] + sq[3]))
            plsc.addupdate_scatter(acc, [idx0 + 33], ones)
            return c
        lax.fori_loop(0, (TH * TW) // NLANE, group, 0)

    start(0, fb0, lb0, s0, s1)
    start(1, fb1, lb1, s2, s3)

    def step(t, c):
        i0 = t * 2
        wait(fb0, lb0, s0, s1)
        compute(fb0, lb0)

        @pl.when(i0 + 2 < nchunk)
        def _():
            start(i0 + 2, fb0, lb0, s0, s1)

        wait(fb1, lb1, s2, s3)
        compute(fb1, lb1)

        @pl.when(i0 + 3 < nchunk)
        def _():
            start(i0 + 3, fb1, lb1, s2, s3)
        return c
    lax.fori_loop(0, nchunk // 2, step, 0)

    def fold(j, c):
        v = acc[pl.ds(j * NLANE, NLANE)]
        for b in range(1, NBANK):
            v = v + acc[pl.ds(b * BSTRIDE + j * NLANE, NLANE)]
        red[j // TH, pl.ds((j % TH) * NLANE, NLANE)] = v
        return c
    lax.fori_loop(0, SEG // NLANE, fold, 0)
    pltpu.sync_copy(red, out.at[wid])


def _epilogue_body(p_ref, var_ref, dist_ref, norm_ref, tot_ref, mu_ref):
    p = p_ref[...]
    N = mu_ref.shape[0]
    wpi = NW // N
    var_acc = jnp.zeros((K, 1), jnp.float32)
    norm_acc = jnp.zeros((K, 1), jnp.float32)
    hs = jnp.zeros((K, K), jnp.float32)
    for n in range(N):
        pn = p[n * wpi]
        for j in range(1, wpi):
            pn = pn + p[n * wpi + j]
        sums = pn[:, :32]
        sumsq = pn[:, 32:33]
        cnt = jnp.maximum(pn[:, 33:34], 1.0)
        mu = sums / cnt
        mu_ref[n, :, :] = mu
        musq = jnp.sum(mu * mu, axis=1, keepdims=True)
        seg_sq = sumsq - cnt * musq
        mse = seg_sq / (32.0 * cnt)
        var_acc = var_acc + jnp.maximum(mse - DELTA_V, 0.0)
        norm_acc = norm_acc + jnp.sqrt(musq + 1e-12)
        diff = mu[:, None, :] - mu[None, :, :]
        d = jnp.sqrt(jnp.sum(diff * diff, axis=2) + 1e-12)
        hs = hs + jnp.maximum(2.0 * DELTA_D - d, 0.0)

    denom = float(N * K)
    var = jnp.sum(var_acc) / denom
    norm = jnp.sum(norm_acc) / denom
    kk = lax.broadcasted_iota(jnp.int32, (K, K), 0)
    ii = lax.broadcasted_iota(jnp.int32, (K, K), 1)
    shift = (kk == ii + 1).astype(jnp.float32)
    hshift = jnp.dot(hs, shift, preferred_element_type=jnp.float32)
    m1 = (kk > ii).astype(jnp.float32)
    m2 = (kk <= ii).astype(jnp.float32)
    dl = jnp.sum(hs * m1 + hshift * m2, axis=0, keepdims=True) / denom

    var_ref[...] = jnp.reshape(var, (1, 1))
    norm_ref[...] = jnp.reshape(norm, (1, 1))
    dist_ref[...] = dl
    tot_ref[...] = ALPHA * var + BETA * dl + GAMMA * norm


def kernel(features, ground_truth):
    N, C, H, W = features.shape

    mesh = plsc.VectorSubcoreMesh(core_axis_name="c", subcore_axis_name="s")
    partials = pl.kernel(
        _sc_body,
        out_type=jax.ShapeDtypeStruct((NW, TH, TW), jnp.float32),
        mesh=mesh,
        compiler_params=pltpu.CompilerParams(
            needs_layout_passes=False, use_tc_tiling_on_sc=True),
        scratch_types=[
            pltpu.VMEM((ACCW,), jnp.float32),
            pltpu.VMEM((TH, TW), jnp.float32),
            pltpu.VMEM((NLANE,), jnp.int32),
            pltpu.VMEM((C, TH, TW), jnp.float32),
            pltpu.VMEM((C, TH, TW), jnp.float32),
            pltpu.VMEM((TH, TW), jnp.int32),
            pltpu.VMEM((TH, TW), jnp.int32),
            pltpu.SemaphoreType.DMA,
            pltpu.SemaphoreType.DMA,
            pltpu.SemaphoreType.DMA,
            pltpu.SemaphoreType.DMA,
        ],
    )(features, ground_truth)

    p3 = partials.reshape(NW, K, SLOTS)
    var, dist, norm, tot, mu = pl.pallas_call(
        _epilogue_body,
        out_shape=(
            jax.ShapeDtypeStruct((1, 1), jnp.float32),
            jax.ShapeDtypeStruct((1, K), jnp.float32),
            jax.ShapeDtypeStruct((1, 1), jnp.float32),
            jax.ShapeDtypeStruct((1, K), jnp.float32),
            jax.ShapeDtypeStruct((N, K, C), jnp.float32),
        ),
    )(p3)

    total_loss = tot[0, : K - 1]
    variance_loss = var[0, 0]
    distance_loss = dist[0, : K - 1]
    normalization_loss = norm[0, 0]
    cluster_mean = jnp.swapaxes(mu, 1, 2)
    return (total_loss, (variance_loss, distance_loss,
                         normalization_loss, cluster_mean))


# speedup vs baseline: 76.6651x; 1.0150x over previous
import functools

import jax
import jax.numpy as jnp
from jax import lax
from jax.experimental import pallas as pl
from jax.experimental.pallas import tpu as pltpu
from jax.experimental.pallas import tpu_sc as plsc

DELTA_V = 0.2
DELTA_D = 0.2
ALPHA = 1.0
BETA = 1.0
GAMMA = 0.001
K = 16
SLOTS = 64
SEG = K * SLOTS
NLANE = 16
NBANK = NLANE
BSTRIDE = SEG + 1
ACCW = 16512
NW = 32
TH, TW = 8, 128


def _sc_body(feat, lbl, out, acc, red, lbv, fb0, fb1, lb0, lb1,
             s0, s1, s2, s3):
    cid = lax.axis_index("c")
    sid = lax.axis_index("s")
    wid = sid * 2 + cid
    N, C, H, W = feat.shape
    wpi = NW // N
    rows = H // wpi
    ntr, ntc = rows // TH, W // TW
    nchunk = ntr * ntc
    n = wid // wpi
    h_base = (wid % wpi) * rows

    zeros = jnp.zeros((NLANE,), jnp.float32)
    ones = jnp.ones((NLANE,), jnp.float32)
    lane = lax.iota(jnp.int32, NLANE)
    lbv[pl.ds(0, NLANE)] = lane * BSTRIDE

    def zero_body(j, c):
        acc[pl.ds(j * NLANE, NLANE)] = zeros
        return c
    lax.fori_loop(0, ACCW // NLANE, zero_body, 0)

    def start(idx, fb, lb, sf, sl):
        h0 = h_base + (idx // ntc) * TH
        w0 = (idx % ntc) * TW
        pltpu.async_copy(feat.at[n, :, pl.ds(h0, TH), pl.ds(w0, TW)], fb, sf)
        pltpu.async_copy(lbl.at[n, pl.ds(h0, TH), pl.ds(w0, TW)], lb, sl)

    def wait(fb, lb, sf, sl):
        pltpu.make_async_copy(
            feat.at[n, :, pl.ds(h_base, TH), pl.ds(0, TW)], fb, sf).wait()
        pltpu.make_async_copy(
            lbl.at[n, pl.ds(h_base, TH), pl.ds(0, TW)], lb, sl).wait()

    def compute(fb, lb):
        @plsc.parallel_loop(0, (TH * TW) // NLANE, 1, unroll=2)
        def group(g):
            r = g // (TW // NLANE)
            col = (g % (TW // NLANE)) * NLANE
            lab = lb[r, pl.ds(col, NLANE)]
            idx0 = lbv[pl.ds(0, NLANE)] + lab * SLOTS
            sq = [zeros, zeros, zeros, zeros]
            pre = 8
            vs = [fb[ch, r, pl.ds(col, NLANE)] for ch in range(pre)] + [None] * (C - pre)
            for ch in range(C):
                if ch + pre < C:
                    vs[ch + pre] = fb[ch + pre, r, pl.ds(col, NLANE)]
                sq[ch % 4] = sq[ch % 4] + vs[ch] * vs[ch]
                plsc.addupdate_scatter(acc, [idx0 + ch], vs[ch])
            plsc.addupdate_scatter(acc, [idx0 + 32], (sq[0] + sq[1]) + (sq[2] + sq[3]))
            plsc.addupdate_scatter(acc, [idx0 + 33], ones)

    start(0, fb0, lb0, s0, s1)
    start(1, fb1, lb1, s2, s3)

    def step(t, c):
        i0 = t * 2
        wait(fb0, lb0, s0, s1)
        compute(fb0, lb0)

        @pl.when(i0 + 2 < nchunk)
        def _():
            start(i0 + 2, fb0, lb0, s0, s1)

        wait(fb1, lb1, s2, s3)
        compute(fb1, lb1)

        @pl.when(i0 + 3 < nchunk)
        def _():
            start(i0 + 3, fb1, lb1, s2, s3)
        return c
    lax.fori_loop(0, nchunk // 2, step, 0)

    def fold(j, c):
        v = acc[pl.ds(j * NLANE, NLANE)]
        for b in range(1, NBANK):
            v = v + acc[pl.ds(b * BSTRIDE + j * NLANE, NLANE)]
        red[j // TH, pl.ds((j % TH) * NLANE, NLANE)] = v
        return c
    lax.fori_loop(0, SEG // NLANE, fold, 0)
    pltpu.sync_copy(red, out.at[wid])


def _epilogue_body(p_ref, var_ref, dist_ref, norm_ref, tot_ref, mu_ref):
    p = p_ref[...]
    N = mu_ref.shape[0]
    wpi = NW // N
    var_acc = jnp.zeros((K, 1), jnp.float32)
    norm_acc = jnp.zeros((K, 1), jnp.float32)
    hs = jnp.zeros((K, K), jnp.float32)
    for n in range(N):
        pn = p[n * wpi]
        for j in range(1, wpi):
            pn = pn + p[n * wpi + j]
        sums = pn[:, :32]
        sumsq = pn[:, 32:33]
        cnt = jnp.maximum(pn[:, 33:34], 1.0)
        mu = sums / cnt
        mu_ref[n, :, :] = mu
        musq = jnp.sum(mu * mu, axis=1, keepdims=True)
        seg_sq = sumsq - cnt * musq
        mse = seg_sq / (32.0 * cnt)
        var_acc = var_acc + jnp.maximum(mse - DELTA_V, 0.0)
        norm_acc = norm_acc + jnp.sqrt(musq + 1e-12)
        diff = mu[:, None, :] - mu[None, :, :]
        d = jnp.sqrt(jnp.sum(diff * diff, axis=2) + 1e-12)
        hs = hs + jnp.maximum(2.0 * DELTA_D - d, 0.0)

    denom = float(N * K)
    var = jnp.sum(var_acc) / denom
    norm = jnp.sum(norm_acc) / denom
    kk = lax.broadcasted_iota(jnp.int32, (K, K), 0)
    ii = lax.broadcasted_iota(jnp.int32, (K, K), 1)
    shift = (kk == ii + 1).astype(jnp.float32)
    hshift = jnp.dot(hs, shift, preferred_element_type=jnp.float32)
    m1 = (kk > ii).astype(jnp.float32)
    m2 = (kk <= ii).astype(jnp.float32)
    dl = jnp.sum(hs * m1 + hshift * m2, axis=0, keepdims=True) / denom

    var_ref[...] = jnp.reshape(var, (1, 1))
    norm_ref[...] = jnp.reshape(norm, (1, 1))
    dist_ref[...] = dl
    tot_ref[...] = ALPHA * var + BETA * dl + GAMMA * norm


def kernel(features, ground_truth):
    N, C, H, W = features.shape

    mesh = plsc.VectorSubcoreMesh(core_axis_name="c", subcore_axis_name="s")
    partials = pl.kernel(
        _sc_body,
        out_type=jax.ShapeDtypeStruct((NW, TH, TW), jnp.float32),
        mesh=mesh,
        compiler_params=pltpu.CompilerParams(
            needs_layout_passes=False, use_tc_tiling_on_sc=True),
        scratch_types=[
            pltpu.VMEM((ACCW,), jnp.float32),
            pltpu.VMEM((TH, TW), jnp.float32),
            pltpu.VMEM((NLANE,), jnp.int32),
            pltpu.VMEM((C, TH, TW), jnp.float32),
            pltpu.VMEM((C, TH, TW), jnp.float32),
            pltpu.VMEM((TH, TW), jnp.int32),
            pltpu.VMEM((TH, TW), jnp.int32),
            pltpu.SemaphoreType.DMA,
            pltpu.SemaphoreType.DMA,
            pltpu.SemaphoreType.DMA,
            pltpu.SemaphoreType.DMA,
        ],
    )(features, ground_truth)

    p3 = partials.reshape(NW, K, SLOTS)
    var, dist, norm, tot, mu = pl.pallas_call(
        _epilogue_body,
        out_shape=(
            jax.ShapeDtypeStruct((1, 1), jnp.float32),
            jax.ShapeDtypeStruct((1, K), jnp.float32),
            jax.ShapeDtypeStruct((1, 1), jnp.float32),
            jax.ShapeDtypeStruct((1, K), jnp.float32),
            jax.ShapeDtypeStruct((N, K, C), jnp.float32),
        ),
    )(p3)

    total_loss = tot[0, : K - 1]
    variance_loss = var[0, 0]
    distance_loss = dist[0, : K - 1]
    normalization_loss = norm[0, 0]
    cluster_mean = jnp.swapaxes(mu, 1, 2)
    return (total_loss, (variance_loss, distance_loss,
                         normalization_loss, cluster_mean))
